# baseline (speedup 1.0000x reference)
import dataclasses
import functools

import jax
import jax.numpy as jnp
from jax import lax
from jax.experimental import pallas as pl
from jax.experimental.pallas import tpu as pltpu
from jax.experimental.pallas import tpu_sc as plsc

MAX_NORM = 1.0 - 0.0001
MAX_NORM_SQ = MAX_NORM * MAX_NORM
LANES = 16
NUM_CORES = 2
NUM_SUBCORES = 16
NUM_WORKERS = NUM_CORES * NUM_SUBCORES


def _row_chunks(m):
    assert m % LANES == 0
    return [pl.ds(j * LANES, LANES) for j in range(m // LANES)]


@functools.partial(jax.jit, static_argnames=("b", "m", "chunk"))
def _sc_embed(x_flat, weight, *, b, m, chunk):
    per_w = b // NUM_WORKERS
    n_chunks = per_w // chunk
    slices = _row_chunks(m)

    mesh = plsc.VectorSubcoreMesh(core_axis_name="c", subcore_axis_name="s")

    cparams = pltpu.CompilerParams()
    if "needs_layout_passes" in pltpu.CompilerParams.__dataclass_fields__:
        cparams = dataclasses.replace(cparams, needs_layout_passes=False)
    if "use_tc_tiling_on_sc" in pltpu.CompilerParams.__dataclass_fields__:
        cparams = dataclasses.replace(cparams, use_tc_tiling_on_sc=False)

    @functools.partial(
        pl.kernel,
        out_type=jax.ShapeDtypeStruct((b, m), jnp.float32),
        mesh=mesh,
        compiler_params=cparams,
        scratch_types=[
            pltpu.VMEM((chunk,), jnp.int32),
            pltpu.VMEM((chunk, m), jnp.float32),
            pltpu.SemaphoreType.DMA,
        ],
    )
    def body(x_hbm, w_hbm, out_hbm, idx_v, rows_v, sem):
        wid = lax.axis_index("s") * NUM_CORES + lax.axis_index("c")
        base = wid * per_w

        for k in range(n_chunks):
            off = base + k * chunk
            pltpu.sync_copy(x_hbm.at[pl.ds(off, chunk)], idx_v)
            pltpu.async_copy(w_hbm.at[idx_v], rows_v, sem).wait()

            def scan_row(r, gmax):
                p = jnp.zeros((LANES,), jnp.float32)
                for sl in slices:
                    v = rows_v[r, sl]
                    p = p + v * v
                return jnp.maximum(gmax, p)

            gmax = lax.fori_loop(
                0, chunk, scan_row, jnp.zeros((LANES,), jnp.float32)
            )
            bound = jnp.sum(gmax)

            @pl.when(bound > MAX_NORM_SQ)
            def _():
                def fix_row(r, carry):
                    vs = [rows_v[r, sl] for sl in slices]
                    p = jnp.zeros((LANES,), jnp.float32)
                    for v in vs:
                        p = p + v * v
                    s2 = jnp.sum(p)
                    s2v = lax.broadcast(s2, (LANES,))
                    bits = lax.bitcast_convert_type(s2v, jnp.int32)
                    y = lax.bitcast_convert_type(
                        0x5F3759DF - (bits >> 1), jnp.float32
                    )
                    for _ in range(3):
                        y = y * (1.5 - 0.5 * s2v * y * y)
                    norm = s2v * y
                    scale = jnp.where(
                        s2v > MAX_NORM_SQ,
                        MAX_NORM / (norm + 1e-7),
                        jnp.float32(1.0),
                    )
                    for sl, v in zip(slices, vs):
                        rows_v[r, sl] = v * scale
                    return carry

                lax.fori_loop(0, chunk, fix_row, 0)

            pltpu.sync_copy(rows_v, out_hbm.at[pl.ds(off, chunk)])

    return body(x_flat, weight)


def kernel(x, weight):
    bsz, hist = x.shape
    n, m = weight.shape
    b = bsz * hist
    x_flat = x.reshape(b).astype(jnp.int32)
    out = _sc_embed(x_flat, weight, b=b, m=m, chunk=640)
    return out.reshape(bsz, hist, m)
